# baseline (speedup 1.0000x reference)
import functools

import jax
import jax.numpy as jnp
from jax import lax
from jax.experimental import pallas as pl
from jax.experimental.pallas import tpu as pltpu
from jax.experimental.pallas import tpu_sc as plsc

N = 50000
E = 800000
F_IN = 80
FH = 40
FP = 64
HD = 64

NC = 2
NS = 16
NW = NC * NS
K = 128
CPAD = 6400
EPAD = CPAD * K
IB2 = 4
IB3 = 8

ROWS_T = 3200
ROWS_LAST = N - 15 * ROWS_T
ZR = 400

f32 = jnp.float32
bf16 = jnp.bfloat16
i32 = jnp.int32

_MESH = plsc.VectorSubcoreMesh(core_axis_name="c", subcore_axis_name="s")
_SC_PARAMS = pltpu.CompilerParams(use_tc_tiling_on_sc=False,
                                  needs_layout_passes=False)


def _zero_vec_buf(ref, nelem):
    def body(g, carry):
        ref[pl.ds(g * 16, 16)] = jnp.zeros((16,), f32)
        return carry
    lax.fori_loop(0, nelem // 16, body, 0)


def _zero_row_buf(ref, nrows):
    def body(r, carry):
        z = jnp.zeros((32,), bf16)
        ref[r, pl.ds(0, 32)] = z
        ref[r, pl.ds(32, 32)] = z
        return carry
    lax.fori_loop(0, nrows, body, 0)


@functools.partial(
    pl.kernel,
    out_type=(jax.ShapeDtypeStruct((N,), f32),
              jax.ShapeDtypeStruct((N,), f32)),
    mesh=_MESH,
    compiler_params=_SC_PARAMS,
    scratch_types=[
        pltpu.VMEM((IB3, K), i32),
        pltpu.VMEM((IB3, K), f32),
        pltpu.VMEM((ZR,), f32),
        pltpu.VMEM_SHARED((N + 16,), f32),
        pltpu.SemaphoreType.DMA((IB3,)),
    ],
)
def _sc_deg(dst_hbm, ew_hbm, p0_hbm, p1_hbm, idx_b, ew_b, zbuf, deg_sh, sems):
    c = lax.axis_index("c")
    s = lax.axis_index("s")
    w = s * NC + c

    _zero_vec_buf(zbuf, ZR)

    @pl.when(s < 15)
    def _():
        for r in range(ROWS_T // ZR):
            pltpu.sync_copy(zbuf, deg_sh.at[pl.ds(s * ROWS_T + r * ZR, ZR)])

    @pl.when(s == 15)
    def _():
        for r in range(ROWS_LAST // ZR):
            pltpu.sync_copy(zbuf, deg_sh.at[pl.ds(15 * ROWS_T + r * ZR, ZR)])

    plsc.subcore_barrier()

    cpw = CPAD // NW

    def body(b, carry):
        base = w * cpw + b * IB3
        pltpu.sync_copy(dst_hbm.at[pl.ds(base, IB3)], idx_b)
        pltpu.sync_copy(ew_hbm.at[pl.ds(base, IB3)], ew_b)
        descs = [
            pltpu.async_copy(ew_b.at[t], deg_sh.at[idx_b.at[t]],
                             sems.at[t], add=True)
            for t in range(IB3)
        ]
        for d in descs:
            d.wait()
        return carry

    lax.fori_loop(0, cpw // IB3, body, 0)
    plsc.subcore_barrier()

    def writeout(out_hbm):
        def wchunk(base):
            pltpu.sync_copy(deg_sh.at[pl.ds(base, ZR)], zbuf)
            pltpu.sync_copy(zbuf, out_hbm.at[pl.ds(base, ZR)])

        @pl.when(s < 15)
        def _():
            for r in range(ROWS_T // ZR):
                wchunk(s * ROWS_T + r * ZR)

        @pl.when(s == 15)
        def _():
            for r in range(ROWS_LAST // ZR):
                wchunk(15 * ROWS_T + r * ZR)

    @pl.when(c == 0)
    def _():
        writeout(p0_hbm)

    @pl.when(c == 1)
    def _():
        writeout(p1_hbm)


@functools.partial(
    pl.kernel,
    out_type=(jax.ShapeDtypeStruct((N, FP), bf16),
              jax.ShapeDtypeStruct((N, FP), bf16)),
    mesh=_MESH,
    compiler_params=_SC_PARAMS,
    scratch_types=[
        pltpu.VMEM((IB2, K), i32),
        pltpu.VMEM((IB2, K), i32),
        pltpu.VMEM((IB2, K), f32),
        pltpu.VMEM((IB2, K, FP), bf16),
        pltpu.VMEM((ZR, FP), bf16),
        pltpu.VMEM_SHARED((N + 16, FP), bf16),
        pltpu.SemaphoreType.DMA((IB2,)),
        pltpu.SemaphoreType.DMA((IB2,)),
    ],
)
def _sc_rows(src_hbm, dst_hbm, ew_hbm, y0_hbm, y1_hbm, s0_hbm, s1_hbm,
             isrc_b, idst_b, ew_b, rows, zrows, s_sh, semg, semsc):
    c = lax.axis_index("c")
    s = lax.axis_index("s")

    _zero_row_buf(zrows, ZR)

    @pl.when(s < 15)
    def _():
        for r in range(ROWS_T // ZR):
            pltpu.sync_copy(zrows, s_sh.at[pl.ds(s * ROWS_T + r * ZR, ZR)])

    @pl.when(s == 15)
    def _():
        for r in range(ROWS_LAST // ZR):
            pltpu.sync_copy(zrows, s_sh.at[pl.ds(15 * ROWS_T + r * ZR, ZR)])

    plsc.subcore_barrier()

    cpt = CPAD // NS

    def pipeline(y_hbm):
        def body(b, carry):
            base = s * cpt + b * IB2
            pltpu.sync_copy(src_hbm.at[pl.ds(base, IB2)], isrc_b)
            pltpu.sync_copy(dst_hbm.at[pl.ds(base, IB2)], idst_b)
            pltpu.sync_copy(ew_hbm.at[pl.ds(base, IB2)], ew_b)
            gd = [
                pltpu.async_copy(y_hbm.at[isrc_b.at[t]], rows.at[t],
                                 semg.at[t])
                for t in range(IB2)
            ]
            sd = []
            for t in range(IB2):
                gd[t].wait()

                def scale(k, carry2, t=t):
                    w16 = plsc.load_gather(
                        ew_b, [jnp.full((16,), t, i32),
                               jnp.full((16,), k, i32)])
                    wb = plsc.pack(w16, w16,
                                   format=plsc.PackFormat.INTERLEAVED)
                    rows[t, k, pl.ds(0, 32)] = rows[t, k, pl.ds(0, 32)] * wb
                    rows[t, k, pl.ds(32, 32)] = rows[t, k, pl.ds(32, 32)] * wb
                    return carry2

                lax.fori_loop(0, K, scale, 0)
                sd.append(pltpu.async_copy(
                    rows.at[t], s_sh.at[idst_b.at[t]], semsc.at[t], add=True))
            for d in sd:
                d.wait()
            return carry

        lax.fori_loop(0, cpt // IB2, body, 0)

    @pl.when(c == 0)
    def _():
        pipeline(y0_hbm)

    @pl.when(c == 1)
    def _():
        pipeline(y1_hbm)

    plsc.subcore_barrier()

    def writeout(out_hbm):
        def wchunk(base):
            pltpu.sync_copy(s_sh.at[pl.ds(base, ZR)], zrows)
            pltpu.sync_copy(zrows, out_hbm.at[pl.ds(base, ZR)])

        @pl.when(s < 15)
        def _():
            for r in range(ROWS_T // ZR):
                wchunk(s * ROWS_T + r * ZR)

        @pl.when(s == 15)
        def _():
            for r in range(ROWS_LAST // ZR):
                wchunk(15 * ROWS_T + r * ZR)

    @pl.when(c == 0)
    def _():
        writeout(s0_hbm)

    @pl.when(c == 1)
    def _():
        writeout(s1_hbm)


@functools.partial(
    pl.kernel,
    out_type=jax.ShapeDtypeStruct((EPAD, HD), f32),
    mesh=_MESH,
    compiler_params=_SC_PARAMS,
    scratch_types=[
        pltpu.VMEM((IB3, K), i32),
        pltpu.VMEM((IB3, K), i32),
        pltpu.VMEM((K, HD), f32),
    ],
)
def _sc_edges(src_hbm, dst_hbm, a_hbm, b_hbm, gs_hbm, isrc_b, idst_b, rows):
    c = lax.axis_index("c")
    s = lax.axis_index("s")
    w = s * NC + c
    cpw = CPAD // NW

    def body(b, carry):
        base = w * cpw + b * IB3
        pltpu.sync_copy(src_hbm.at[pl.ds(base, IB3)], isrc_b)
        pltpu.sync_copy(dst_hbm.at[pl.ds(base, IB3)], idst_b)
        for t in range(IB3):
            pltpu.sync_copy(a_hbm.at[isrc_b.at[t]], rows)
            pltpu.sync_copy(b_hbm.at[idst_b.at[t]], rows, add=True)
            pltpu.sync_copy(rows, gs_hbm.at[pl.ds((base + t) * K, K)])
        return carry

    lax.fori_loop(0, cpw // IB3, body, 0)


_BN = 2000

def _tca_body(p0, p1, x2, dinv, y0, y1):
    deg = p0[...] + p1[...] + 1.0
    dv = lax.rsqrt(deg)
    dinv[...] = dv
    zpad = jnp.zeros((_BN, FP - FH), bf16)
    y0[...] = jnp.concatenate([(dv * x2[:, :FH]).astype(bf16), zpad], axis=1)
    y1[...] = jnp.concatenate([(dv * x2[:, FH:]).astype(bf16), zpad], axis=1)


def _tc_a(p0, p1, x2):
    grid = N // _BN
    return pl.pallas_call(
        _tca_body,
        grid=(grid,),
        in_specs=[
            pl.BlockSpec((_BN, 1), lambda i: (i, 0)),
            pl.BlockSpec((_BN, 1), lambda i: (i, 0)),
            pl.BlockSpec((_BN, F_IN), lambda i: (i, 0)),
        ],
        out_specs=[
            pl.BlockSpec((_BN, 1), lambda i: (i, 0)),
            pl.BlockSpec((_BN, FP), lambda i: (i, 0)),
            pl.BlockSpec((_BN, FP), lambda i: (i, 0)),
        ],
        out_shape=[
            jax.ShapeDtypeStruct((N, 1), f32),
            jax.ShapeDtypeStruct((N, FP), bf16),
            jax.ShapeDtypeStruct((N, FP), bf16),
        ],
    )(p0, p1, x2)


def _tcb_body(s0, s1, x2, dinv, wz, bz, lzw, lzb, wh, bh, lhw, lhb,
              m1w, m1b, a_out, b_out):
    dv = dinv[...]
    dv2 = dv * dv
    ax0 = dv * s0[:, :FH].astype(f32) + dv2 * x2[:, :FH]
    ax1 = dv * s1[:, :FH].astype(f32) + dv2 * x2[:, FH:]
    aggx = jnp.concatenate([ax0, ax1], axis=1)
    hp = jax.lax.Precision.HIGHEST
    wzp = jnp.dot(wz[...], lzw[:HD, :], precision=hp)
    bzp = jnp.dot(bz[...], lzw[:HD, :], precision=hp) + lzb[...]
    whp = jnp.dot(wh[...], lhw[:HD, :], precision=hp)
    bhp = jnp.dot(bh[...], lhw[:HD, :], precision=hp) + lhb[...]
    z = jax.nn.sigmoid(jnp.dot(aggx, wzp, precision=hp) + bzp)
    ht = jnp.tanh(jnp.dot(aggx, whp, precision=hp) + bhp)
    h = (1.0 - z) * ht
    a_out[...] = jnp.dot(h, m1w[:HD, :], precision=hp) + m1b[...]
    b_out[...] = jnp.dot(h, m1w[HD:, :], precision=hp)


def _tc_b(s0, s1, x2, dinv, wz, bz, lzw, lzb, wh, bh, lhw, lhb, m1w, m1b):
    grid = N // _BN
    full = lambda shape: pl.BlockSpec(shape, lambda i: tuple(0 for _ in shape))
    return pl.pallas_call(
        _tcb_body,
        grid=(grid,),
        in_specs=[
            pl.BlockSpec((_BN, FP), lambda i: (i, 0)),
            pl.BlockSpec((_BN, FP), lambda i: (i, 0)),
            pl.BlockSpec((_BN, F_IN), lambda i: (i, 0)),
            pl.BlockSpec((_BN, 1), lambda i: (i, 0)),
            full((F_IN, HD)), full((1, HD)), full((2 * HD, HD)), full((1, HD)),
            full((F_IN, HD)), full((1, HD)), full((2 * HD, HD)), full((1, HD)),
            full((2 * HD, HD)), full((1, HD)),
        ],
        out_specs=[
            pl.BlockSpec((_BN, HD), lambda i: (i, 0)),
            pl.BlockSpec((_BN, HD), lambda i: (i, 0)),
        ],
        out_shape=[
            jax.ShapeDtypeStruct((N, HD), f32),
            jax.ShapeDtypeStruct((N, HD), f32),
        ],
    )(s0, s1, x2, dinv, wz, bz, lzw, lzb, wh, bh, lhw, lhb, m1w, m1b)


_BE = 2000

def _tcc_body(gs, m2w, m2b, out):
    r = jnp.maximum(gs[...], 0.0)
    out[...] = jnp.dot(r, m2w[...],
                       precision=jax.lax.Precision.HIGHEST) + m2b[...]


def _tc_c(gs, m2w, m2b):
    grid = E // _BE
    return pl.pallas_call(
        _tcc_body,
        grid=(grid,),
        in_specs=[
            pl.BlockSpec((_BE, HD), lambda i: (i, 0)),
            pl.BlockSpec((HD, 2), lambda i: (0, 0)),
            pl.BlockSpec((1, 2), lambda i: (0, 0)),
        ],
        out_specs=pl.BlockSpec((_BE, 2), lambda i: (i, 0)),
        out_shape=jax.ShapeDtypeStruct((E, 2), f32),
    )(gs, m2w, m2b)


def kernel(x, edge_index, edge_weight, edge_src, edge_dst,
           W_z, b_z, Lz_W, Lz_b, W_r, b_r, Lr_W, Lr_b, W_h, b_h, Lh_W, Lh_b,
           att, M1_W, M1_b, M2_W, M2_b):
    x2 = x[0, :, :, 0]
    src = edge_index[0]
    dst = edge_index[1]

    pad = EPAD - E
    zpad_i = jnp.zeros((pad,), i32)
    src2d = jnp.concatenate([src, zpad_i]).reshape(CPAD, K)
    dst2d = jnp.concatenate([dst, jnp.full((pad,), N, i32)]).reshape(CPAD, K)
    ew2d = jnp.concatenate([edge_weight,
                            jnp.zeros((pad,), f32)]).reshape(CPAD, K)
    esrc2d = jnp.concatenate([edge_src, zpad_i]).reshape(CPAD, K)
    edst2d = jnp.concatenate([edge_dst, zpad_i]).reshape(CPAD, K)

    p0, p1 = _sc_deg(dst2d, ew2d)
    dinv, y0, y1 = _tc_a(p0.reshape(N, 1), p1.reshape(N, 1), x2)
    s0, s1 = _sc_rows(src2d, dst2d, ew2d, y0, y1)
    a_tab, b_tab = _tc_b(
        s0, s1, x2, dinv,
        W_z, b_z.reshape(1, HD), Lz_W, Lz_b.reshape(1, HD),
        W_h, b_h.reshape(1, HD), Lh_W, Lh_b.reshape(1, HD),
        M1_W, M1_b.reshape(1, HD))
    gs = _sc_edges(esrc2d, edst2d, a_tab, b_tab)
    return _tc_c(gs[:E], M2_W, M2_b.reshape(1, 2))


# speedup vs baseline: 1.1525x; 1.1525x over previous
import functools

import jax
import jax.numpy as jnp
from jax import lax
from jax.experimental import pallas as pl
from jax.experimental.pallas import tpu as pltpu
from jax.experimental.pallas import tpu_sc as plsc

N = 50000
E = 800000
F_IN = 80
FH = 40
FP = 64
HD = 64

NC = 2
NS = 16
NW = NC * NS
K = 128
CPAD = 6400
EPAD = CPAD * K
IB2 = 4
IB3 = 8

ROWS_T = 3200
ROWS_LAST = N - 15 * ROWS_T
ZR = 400

f32 = jnp.float32
bf16 = jnp.bfloat16
i32 = jnp.int32

_MESH = plsc.VectorSubcoreMesh(core_axis_name="c", subcore_axis_name="s")
_SC_PARAMS = pltpu.CompilerParams(use_tc_tiling_on_sc=False,
                                  needs_layout_passes=False)


def _zero_vec_buf(ref, nelem):
    def body(g, carry):
        ref[pl.ds(g * 16, 16)] = jnp.zeros((16,), f32)
        return carry
    lax.fori_loop(0, nelem // 16, body, 0)


def _zero_row_buf(ref, nrows):
    def body(r, carry):
        z = jnp.zeros((32,), bf16)
        ref[r, pl.ds(0, 32)] = z
        ref[r, pl.ds(32, 32)] = z
        return carry
    lax.fori_loop(0, nrows, body, 0)


@functools.partial(
    pl.kernel,
    out_type=(jax.ShapeDtypeStruct((N,), f32),
              jax.ShapeDtypeStruct((N,), f32)),
    mesh=_MESH,
    compiler_params=_SC_PARAMS,
    scratch_types=[
        pltpu.VMEM((IB3, K), i32),
        pltpu.VMEM((IB3, K), f32),
        pltpu.VMEM((ZR,), f32),
        pltpu.VMEM_SHARED((N + 16,), f32),
        pltpu.SemaphoreType.DMA((IB3,)),
    ],
)
def _sc_deg(dst_hbm, ew_hbm, p0_hbm, p1_hbm, idx_b, ew_b, zbuf, deg_sh, sems):
    c = lax.axis_index("c")
    s = lax.axis_index("s")
    w = s * NC + c

    _zero_vec_buf(zbuf, ZR)

    @pl.when(s < 15)
    def _():
        for r in range(ROWS_T // ZR):
            pltpu.sync_copy(zbuf, deg_sh.at[pl.ds(s * ROWS_T + r * ZR, ZR)])

    @pl.when(s == 15)
    def _():
        for r in range(ROWS_LAST // ZR):
            pltpu.sync_copy(zbuf, deg_sh.at[pl.ds(15 * ROWS_T + r * ZR, ZR)])

    plsc.subcore_barrier()

    cpw = CPAD // NW

    def body(b, carry):
        base = w * cpw + b * IB3
        pltpu.sync_copy(dst_hbm.at[pl.ds(base, IB3)], idx_b)
        pltpu.sync_copy(ew_hbm.at[pl.ds(base, IB3)], ew_b)
        descs = [
            pltpu.async_copy(ew_b.at[t], deg_sh.at[idx_b.at[t]],
                             sems.at[t], add=True)
            for t in range(IB3)
        ]
        for d in descs:
            d.wait()
        return carry

    lax.fori_loop(0, cpw // IB3, body, 0)
    plsc.subcore_barrier()

    def writeout(out_hbm):
        def wchunk(base):
            pltpu.sync_copy(deg_sh.at[pl.ds(base, ZR)], zbuf)
            pltpu.sync_copy(zbuf, out_hbm.at[pl.ds(base, ZR)])

        @pl.when(s < 15)
        def _():
            for r in range(ROWS_T // ZR):
                wchunk(s * ROWS_T + r * ZR)

        @pl.when(s == 15)
        def _():
            for r in range(ROWS_LAST // ZR):
                wchunk(15 * ROWS_T + r * ZR)

    @pl.when(c == 0)
    def _():
        writeout(p0_hbm)

    @pl.when(c == 1)
    def _():
        writeout(p1_hbm)


@functools.partial(
    pl.kernel,
    out_type=(jax.ShapeDtypeStruct((N, FP), bf16),
              jax.ShapeDtypeStruct((N, FP), bf16)),
    mesh=_MESH,
    compiler_params=_SC_PARAMS,
    scratch_types=[
        pltpu.VMEM((IB2, K), i32),
        pltpu.VMEM((IB2, K), i32),
        pltpu.VMEM((IB2, K), f32),
        pltpu.VMEM((IB2, K, FP), bf16),
        pltpu.VMEM((ZR, FP), bf16),
        pltpu.VMEM_SHARED((N + 16, FP), bf16),
        pltpu.SemaphoreType.DMA((IB2,)),
        pltpu.SemaphoreType.DMA((IB2,)),
    ],
)
def _sc_rows(src_hbm, dst_hbm, ew_hbm, y0_hbm, y1_hbm, s0_hbm, s1_hbm,
             isrc_b, idst_b, ew_b, rows, zrows, s_sh, semg, semsc):
    c = lax.axis_index("c")
    s = lax.axis_index("s")

    _zero_row_buf(zrows, ZR)

    @pl.when(s < 15)
    def _():
        for r in range(ROWS_T // ZR):
            pltpu.sync_copy(zrows, s_sh.at[pl.ds(s * ROWS_T + r * ZR, ZR)])

    @pl.when(s == 15)
    def _():
        for r in range(ROWS_LAST // ZR):
            pltpu.sync_copy(zrows, s_sh.at[pl.ds(15 * ROWS_T + r * ZR, ZR)])

    plsc.subcore_barrier()

    cpt = CPAD // NS

    def pipeline(y_hbm):
        def body(b, carry):
            base = s * cpt + b * IB2
            pltpu.sync_copy(src_hbm.at[pl.ds(base, IB2)], isrc_b)
            pltpu.sync_copy(dst_hbm.at[pl.ds(base, IB2)], idst_b)
            pltpu.sync_copy(ew_hbm.at[pl.ds(base, IB2)], ew_b)
            gd = [
                pltpu.async_copy(y_hbm.at[isrc_b.at[t]], rows.at[t],
                                 semg.at[t])
                for t in range(IB2)
            ]
            sd = []
            for t in range(IB2):
                gd[t].wait()

                def scale(k, carry2, t=t):
                    w16 = plsc.load_gather(
                        ew_b, [jnp.full((16,), t, i32),
                               jnp.full((16,), k, i32)])
                    wb = plsc.pack(w16, w16,
                                   format=plsc.PackFormat.INTERLEAVED)
                    rows[t, k, pl.ds(0, 32)] = rows[t, k, pl.ds(0, 32)] * wb
                    rows[t, k, pl.ds(32, 32)] = rows[t, k, pl.ds(32, 32)] * wb
                    return carry2

                lax.fori_loop(0, K, scale, 0)
                sd.append(pltpu.async_copy(
                    rows.at[t], s_sh.at[idst_b.at[t]], semsc.at[t], add=True))
            for d in sd:
                d.wait()
            return carry

        lax.fori_loop(0, cpt // IB2, body, 0)

    @pl.when(c == 0)
    def _():
        pipeline(y0_hbm)

    @pl.when(c == 1)
    def _():
        pipeline(y1_hbm)

    plsc.subcore_barrier()

    def writeout(out_hbm):
        def wchunk(base):
            pltpu.sync_copy(s_sh.at[pl.ds(base, ZR)], zrows)
            pltpu.sync_copy(zrows, out_hbm.at[pl.ds(base, ZR)])

        @pl.when(s < 15)
        def _():
            for r in range(ROWS_T // ZR):
                wchunk(s * ROWS_T + r * ZR)

        @pl.when(s == 15)
        def _():
            for r in range(ROWS_LAST // ZR):
                wchunk(15 * ROWS_T + r * ZR)

    @pl.when(c == 0)
    def _():
        writeout(s0_hbm)

    @pl.when(c == 1)
    def _():
        writeout(s1_hbm)


@functools.partial(
    pl.kernel,
    out_type=jax.ShapeDtypeStruct((EPAD, HD), f32),
    mesh=_MESH,
    compiler_params=_SC_PARAMS,
    scratch_types=[
        pltpu.VMEM((IB3, K), i32),
        pltpu.VMEM((IB3, K), i32),
        pltpu.VMEM((K, HD), f32),
    ],
)
def _sc_edges(src_hbm, dst_hbm, a_hbm, b_hbm, gs_hbm, isrc_b, idst_b, rows):
    c = lax.axis_index("c")
    s = lax.axis_index("s")
    w = s * NC + c

    def body(b, carry):
        base = (w + NW * b) * IB3
        pltpu.sync_copy(src_hbm.at[pl.ds(base, IB3)], isrc_b)
        pltpu.sync_copy(dst_hbm.at[pl.ds(base, IB3)], idst_b)
        for t in range(IB3):
            pltpu.sync_copy(a_hbm.at[isrc_b.at[t]], rows)
            pltpu.sync_copy(b_hbm.at[idst_b.at[t]], rows, add=True)
            pltpu.sync_copy(rows, gs_hbm.at[pl.ds((base + t) * K, K)])
        return carry

    lax.fori_loop(0, CPAD // (NW * IB3), body, 0)


_BN = 2000

def _tca_body(p0, p1, x2, dinv, y0, y1):
    deg = p0[...] + p1[...] + 1.0
    dv = lax.rsqrt(deg)
    dinv[...] = dv
    zpad = jnp.zeros((_BN, FP - FH), bf16)
    y0[...] = jnp.concatenate([(dv * x2[:, :FH]).astype(bf16), zpad], axis=1)
    y1[...] = jnp.concatenate([(dv * x2[:, FH:]).astype(bf16), zpad], axis=1)


def _tc_a(p0, p1, x2):
    grid = N // _BN
    return pl.pallas_call(
        _tca_body,
        grid=(grid,),
        in_specs=[
            pl.BlockSpec((_BN, 1), lambda i: (i, 0)),
            pl.BlockSpec((_BN, 1), lambda i: (i, 0)),
            pl.BlockSpec((_BN, F_IN), lambda i: (i, 0)),
        ],
        out_specs=[
            pl.BlockSpec((_BN, 1), lambda i: (i, 0)),
            pl.BlockSpec((_BN, FP), lambda i: (i, 0)),
            pl.BlockSpec((_BN, FP), lambda i: (i, 0)),
        ],
        out_shape=[
            jax.ShapeDtypeStruct((N, 1), f32),
            jax.ShapeDtypeStruct((N, FP), bf16),
            jax.ShapeDtypeStruct((N, FP), bf16),
        ],
    )(p0, p1, x2)


def _tcb_body(s0, s1, x2, dinv, wz, bz, lzw, lzb, wh, bh, lhw, lhb,
              m1w, m1b, a_out, b_out):
    dv = dinv[...]
    dv2 = dv * dv
    ax0 = dv * s0[:, :FH].astype(f32) + dv2 * x2[:, :FH]
    ax1 = dv * s1[:, :FH].astype(f32) + dv2 * x2[:, FH:]
    aggx = jnp.concatenate([ax0, ax1], axis=1)
    hp = jax.lax.Precision.HIGHEST
    wzp = jnp.dot(wz[...], lzw[:HD, :], precision=hp)
    bzp = jnp.dot(bz[...], lzw[:HD, :], precision=hp) + lzb[...]
    whp = jnp.dot(wh[...], lhw[:HD, :], precision=hp)
    bhp = jnp.dot(bh[...], lhw[:HD, :], precision=hp) + lhb[...]
    z = jax.nn.sigmoid(jnp.dot(aggx, wzp, precision=hp) + bzp)
    ht = jnp.tanh(jnp.dot(aggx, whp, precision=hp) + bhp)
    h = (1.0 - z) * ht
    a_out[...] = jnp.dot(h, m1w[:HD, :], precision=hp) + m1b[...]
    b_out[...] = jnp.dot(h, m1w[HD:, :], precision=hp)


def _tc_b(s0, s1, x2, dinv, wz, bz, lzw, lzb, wh, bh, lhw, lhb, m1w, m1b):
    grid = N // _BN
    full = lambda shape: pl.BlockSpec(shape, lambda i: tuple(0 for _ in shape))
    return pl.pallas_call(
        _tcb_body,
        grid=(grid,),
        in_specs=[
            pl.BlockSpec((_BN, FP), lambda i: (i, 0)),
            pl.BlockSpec((_BN, FP), lambda i: (i, 0)),
            pl.BlockSpec((_BN, F_IN), lambda i: (i, 0)),
            pl.BlockSpec((_BN, 1), lambda i: (i, 0)),
            full((F_IN, HD)), full((1, HD)), full((2 * HD, HD)), full((1, HD)),
            full((F_IN, HD)), full((1, HD)), full((2 * HD, HD)), full((1, HD)),
            full((2 * HD, HD)), full((1, HD)),
        ],
        out_specs=[
            pl.BlockSpec((_BN, HD), lambda i: (i, 0)),
            pl.BlockSpec((_BN, HD), lambda i: (i, 0)),
        ],
        out_shape=[
            jax.ShapeDtypeStruct((N, HD), f32),
            jax.ShapeDtypeStruct((N, HD), f32),
        ],
    )(s0, s1, x2, dinv, wz, bz, lzw, lzb, wh, bh, lhw, lhb, m1w, m1b)


_BE = 2000

def _tcc_body(gs, m2w, m2b, out):
    r = jnp.maximum(gs[...], 0.0)
    out[...] = jnp.dot(r, m2w[...],
                       precision=jax.lax.Precision.HIGHEST) + m2b[...]


def _tc_c(gs, m2w, m2b):
    grid = E // _BE
    return pl.pallas_call(
        _tcc_body,
        grid=(grid,),
        in_specs=[
            pl.BlockSpec((_BE, HD), lambda i: (i, 0)),
            pl.BlockSpec((HD, 2), lambda i: (0, 0)),
            pl.BlockSpec((1, 2), lambda i: (0, 0)),
        ],
        out_specs=pl.BlockSpec((_BE, 2), lambda i: (i, 0)),
        out_shape=jax.ShapeDtypeStruct((E, 2), f32),
    )(gs, m2w, m2b)


def kernel(x, edge_index, edge_weight, edge_src, edge_dst,
           W_z, b_z, Lz_W, Lz_b, W_r, b_r, Lr_W, Lr_b, W_h, b_h, Lh_W, Lh_b,
           att, M1_W, M1_b, M2_W, M2_b):
    x2 = x[0, :, :, 0]
    src = edge_index[0]
    dst = edge_index[1]

    pad = EPAD - E
    zpad_i = jnp.zeros((pad,), i32)
    src2d = jnp.concatenate([src, zpad_i]).reshape(CPAD, K)
    dst2d = jnp.concatenate([dst, jnp.full((pad,), N, i32)]).reshape(CPAD, K)
    ew2d = jnp.concatenate([edge_weight,
                            jnp.zeros((pad,), f32)]).reshape(CPAD, K)
    esrc2d = jnp.concatenate([edge_src, zpad_i]).reshape(CPAD, K)
    edst2d = jnp.concatenate([edge_dst, zpad_i]).reshape(CPAD, K)

    p0, p1 = _sc_deg(dst2d, ew2d)
    dinv, y0, y1 = _tc_a(p0.reshape(N, 1), p1.reshape(N, 1), x2)
    s0, s1 = _sc_rows(src2d, dst2d, ew2d, y0, y1)
    a_tab, b_tab = _tc_b(
        s0, s1, x2, dinv,
        W_z, b_z.reshape(1, HD), Lz_W, Lz_b.reshape(1, HD),
        W_h, b_h.reshape(1, HD), Lh_W, Lh_b.reshape(1, HD),
        M1_W, M1_b.reshape(1, HD))
    gs = _sc_edges(esrc2d, edst2d, a_tab, b_tab)
    return _tc_c(gs[:E], M2_W, M2_b.reshape(1, 2))


# speedup vs baseline: 1.1753x; 1.0198x over previous
import functools

import jax
import jax.numpy as jnp
from jax import lax
from jax.experimental import pallas as pl
from jax.experimental.pallas import tpu as pltpu
from jax.experimental.pallas import tpu_sc as plsc

N = 50000
E = 800000
F_IN = 80
FH = 40
FP = 64
HD = 64

NC = 2
NS = 16
NW = NC * NS
K = 128
CPAD = 6400
EPAD = CPAD * K
IB2 = 4
IB3 = 8

ROWS_T = 3200
ROWS_LAST = N - 15 * ROWS_T
ZR = 400

f32 = jnp.float32
bf16 = jnp.bfloat16
i32 = jnp.int32

_MESH = plsc.VectorSubcoreMesh(core_axis_name="c", subcore_axis_name="s")
_SC_PARAMS = pltpu.CompilerParams(use_tc_tiling_on_sc=False,
                                  needs_layout_passes=False)


def _zero_vec_buf(ref, nelem):
    def body(g, carry):
        ref[pl.ds(g * 16, 16)] = jnp.zeros((16,), f32)
        return carry
    lax.fori_loop(0, nelem // 16, body, 0)


def _zero_row_buf(ref, nrows):
    def body(r, carry):
        z = jnp.zeros((32,), bf16)
        ref[r, pl.ds(0, 32)] = z
        ref[r, pl.ds(32, 32)] = z
        return carry
    lax.fori_loop(0, nrows, body, 0)


@functools.partial(
    pl.kernel,
    out_type=(jax.ShapeDtypeStruct((N,), f32),
              jax.ShapeDtypeStruct((N,), f32)),
    mesh=_MESH,
    compiler_params=_SC_PARAMS,
    scratch_types=[
        pltpu.VMEM((IB3, K), i32),
        pltpu.VMEM((IB3, K), f32),
        pltpu.VMEM((ZR,), f32),
        pltpu.VMEM_SHARED((N + 16,), f32),
        pltpu.SemaphoreType.DMA((IB3,)),
    ],
)
def _sc_deg(dst_hbm, ew_hbm, p0_hbm, p1_hbm, idx_b, ew_b, zbuf, deg_sh, sems):
    c = lax.axis_index("c")
    s = lax.axis_index("s")
    w = s * NC + c

    _zero_vec_buf(zbuf, ZR)

    @pl.when(s < 15)
    def _():
        for r in range(ROWS_T // ZR):
            pltpu.sync_copy(zbuf, deg_sh.at[pl.ds(s * ROWS_T + r * ZR, ZR)])

    @pl.when(s == 15)
    def _():
        for r in range(ROWS_LAST // ZR):
            pltpu.sync_copy(zbuf, deg_sh.at[pl.ds(15 * ROWS_T + r * ZR, ZR)])

    plsc.subcore_barrier()

    def body(b, carry):
        base = (w + NW * b) * IB3
        pltpu.sync_copy(dst_hbm.at[pl.ds(base, IB3)], idx_b)
        pltpu.sync_copy(ew_hbm.at[pl.ds(base, IB3)], ew_b)
        descs = [
            pltpu.async_copy(ew_b.at[t], deg_sh.at[idx_b.at[t]],
                             sems.at[t], add=True)
            for t in range(IB3)
        ]
        for d in descs:
            d.wait()
        return carry

    lax.fori_loop(0, CPAD // (NW * IB3), body, 0)
    plsc.subcore_barrier()

    def writeout(out_hbm):
        def wchunk(base):
            pltpu.sync_copy(deg_sh.at[pl.ds(base, ZR)], zbuf)
            pltpu.sync_copy(zbuf, out_hbm.at[pl.ds(base, ZR)])

        @pl.when(s < 15)
        def _():
            for r in range(ROWS_T // ZR):
                wchunk(s * ROWS_T + r * ZR)

        @pl.when(s == 15)
        def _():
            for r in range(ROWS_LAST // ZR):
                wchunk(15 * ROWS_T + r * ZR)

    @pl.when(c == 0)
    def _():
        writeout(p0_hbm)

    @pl.when(c == 1)
    def _():
        writeout(p1_hbm)


@functools.partial(
    pl.kernel,
    out_type=(jax.ShapeDtypeStruct((N, FP), bf16),
              jax.ShapeDtypeStruct((N, FP), bf16)),
    mesh=_MESH,
    compiler_params=_SC_PARAMS,
    scratch_types=[
        pltpu.VMEM((IB2, K), i32),
        pltpu.VMEM((IB2, K), i32),
        pltpu.VMEM((IB2, K), f32),
        pltpu.VMEM((IB2, K, FP), bf16),
        pltpu.VMEM((ZR, FP), bf16),
        pltpu.VMEM_SHARED((N + 16, FP), bf16),
        pltpu.SemaphoreType.DMA((IB2,)),
        pltpu.SemaphoreType.DMA((IB2,)),
    ],
)
def _sc_rows(src_hbm, dst_hbm, ew_hbm, y0_hbm, y1_hbm, s0_hbm, s1_hbm,
             isrc_b, idst_b, ew_b, rows, zrows, s_sh, semg, semsc):
    c = lax.axis_index("c")
    s = lax.axis_index("s")

    _zero_row_buf(zrows, ZR)

    @pl.when(s < 15)
    def _():
        for r in range(ROWS_T // ZR):
            pltpu.sync_copy(zrows, s_sh.at[pl.ds(s * ROWS_T + r * ZR, ZR)])

    @pl.when(s == 15)
    def _():
        for r in range(ROWS_LAST // ZR):
            pltpu.sync_copy(zrows, s_sh.at[pl.ds(15 * ROWS_T + r * ZR, ZR)])

    plsc.subcore_barrier()

    def pipeline(y_hbm):
        def body(b, carry):
            base = (s + NS * b) * IB2
            pltpu.sync_copy(src_hbm.at[pl.ds(base, IB2)], isrc_b)
            pltpu.sync_copy(dst_hbm.at[pl.ds(base, IB2)], idst_b)
            pltpu.sync_copy(ew_hbm.at[pl.ds(base, IB2)], ew_b)
            gd = [
                pltpu.async_copy(y_hbm.at[isrc_b.at[t]], rows.at[t],
                                 semg.at[t])
                for t in range(IB2)
            ]
            sd = []
            for t in range(IB2):
                gd[t].wait()

                def scale(k, carry2, t=t):
                    w16 = plsc.load_gather(
                        ew_b, [jnp.full((16,), t, i32),
                               jnp.full((16,), k, i32)])
                    wb = plsc.pack(w16, w16,
                                   format=plsc.PackFormat.INTERLEAVED)
                    rows[t, k, pl.ds(0, 32)] = rows[t, k, pl.ds(0, 32)] * wb
                    rows[t, k, pl.ds(32, 32)] = rows[t, k, pl.ds(32, 32)] * wb
                    return carry2

                lax.fori_loop(0, K, scale, 0)
                sd.append(pltpu.async_copy(
                    rows.at[t], s_sh.at[idst_b.at[t]], semsc.at[t], add=True))
            for d in sd:
                d.wait()
            return carry

        lax.fori_loop(0, CPAD // (NS * IB2), body, 0)

    @pl.when(c == 0)
    def _():
        pipeline(y0_hbm)

    @pl.when(c == 1)
    def _():
        pipeline(y1_hbm)

    plsc.subcore_barrier()

    def writeout(out_hbm):
        def wchunk(base):
            pltpu.sync_copy(s_sh.at[pl.ds(base, ZR)], zrows)
            pltpu.sync_copy(zrows, out_hbm.at[pl.ds(base, ZR)])

        @pl.when(s < 15)
        def _():
            for r in range(ROWS_T // ZR):
                wchunk(s * ROWS_T + r * ZR)

        @pl.when(s == 15)
        def _():
            for r in range(ROWS_LAST // ZR):
                wchunk(15 * ROWS_T + r * ZR)

    @pl.when(c == 0)
    def _():
        writeout(s0_hbm)

    @pl.when(c == 1)
    def _():
        writeout(s1_hbm)


@functools.partial(
    pl.kernel,
    out_type=jax.ShapeDtypeStruct((EPAD, HD), f32),
    mesh=_MESH,
    compiler_params=_SC_PARAMS,
    scratch_types=[
        pltpu.VMEM((IB3, K), i32),
        pltpu.VMEM((IB3, K), i32),
        pltpu.VMEM((K, HD), f32),
    ],
)
def _sc_edges(src_hbm, dst_hbm, a_hbm, b_hbm, gs_hbm, isrc_b, idst_b, rows):
    c = lax.axis_index("c")
    s = lax.axis_index("s")
    w = s * NC + c

    def body(b, carry):
        base = (w + NW * b) * IB3
        pltpu.sync_copy(src_hbm.at[pl.ds(base, IB3)], isrc_b)
        pltpu.sync_copy(dst_hbm.at[pl.ds(base, IB3)], idst_b)
        for t in range(IB3):
            pltpu.sync_copy(a_hbm.at[isrc_b.at[t]], rows)
            pltpu.sync_copy(b_hbm.at[idst_b.at[t]], rows, add=True)
            pltpu.sync_copy(rows, gs_hbm.at[pl.ds((base + t) * K, K)])
        return carry

    lax.fori_loop(0, CPAD // (NW * IB3), body, 0)


_BN = 2000

def _tca_body(p0, p1, x2, dinv, y0, y1):
    deg = p0[...] + p1[...] + 1.0
    dv = lax.rsqrt(deg)
    dinv[...] = dv
    zpad = jnp.zeros((_BN, FP - FH), bf16)
    y0[...] = jnp.concatenate([(dv * x2[:, :FH]).astype(bf16), zpad], axis=1)
    y1[...] = jnp.concatenate([(dv * x2[:, FH:]).astype(bf16), zpad], axis=1)


def _tc_a(p0, p1, x2):
    grid = N // _BN
    return pl.pallas_call(
        _tca_body,
        grid=(grid,),
        in_specs=[
            pl.BlockSpec((_BN, 1), lambda i: (i, 0)),
            pl.BlockSpec((_BN, 1), lambda i: (i, 0)),
            pl.BlockSpec((_BN, F_IN), lambda i: (i, 0)),
        ],
        out_specs=[
            pl.BlockSpec((_BN, 1), lambda i: (i, 0)),
            pl.BlockSpec((_BN, FP), lambda i: (i, 0)),
            pl.BlockSpec((_BN, FP), lambda i: (i, 0)),
        ],
        out_shape=[
            jax.ShapeDtypeStruct((N, 1), f32),
            jax.ShapeDtypeStruct((N, FP), bf16),
            jax.ShapeDtypeStruct((N, FP), bf16),
        ],
    )(p0, p1, x2)


def _tcb_body(s0, s1, x2, dinv, wz, bz, lzw, lzb, wh, bh, lhw, lhb,
              m1w, m1b, a_out, b_out):
    dv = dinv[...]
    dv2 = dv * dv
    ax0 = dv * s0[:, :FH].astype(f32) + dv2 * x2[:, :FH]
    ax1 = dv * s1[:, :FH].astype(f32) + dv2 * x2[:, FH:]
    aggx = jnp.concatenate([ax0, ax1], axis=1)
    hp = jax.lax.Precision.HIGHEST
    wzp = jnp.dot(wz[...], lzw[:HD, :], precision=hp)
    bzp = jnp.dot(bz[...], lzw[:HD, :], precision=hp) + lzb[...]
    whp = jnp.dot(wh[...], lhw[:HD, :], precision=hp)
    bhp = jnp.dot(bh[...], lhw[:HD, :], precision=hp) + lhb[...]
    z = jax.nn.sigmoid(jnp.dot(aggx, wzp, precision=hp) + bzp)
    ht = jnp.tanh(jnp.dot(aggx, whp, precision=hp) + bhp)
    h = (1.0 - z) * ht
    a_out[...] = jnp.dot(h, m1w[:HD, :], precision=hp) + m1b[...]
    b_out[...] = jnp.dot(h, m1w[HD:, :], precision=hp)


def _tc_b(s0, s1, x2, dinv, wz, bz, lzw, lzb, wh, bh, lhw, lhb, m1w, m1b):
    grid = N // _BN
    full = lambda shape: pl.BlockSpec(shape, lambda i: tuple(0 for _ in shape))
    return pl.pallas_call(
        _tcb_body,
        grid=(grid,),
        in_specs=[
            pl.BlockSpec((_BN, FP), lambda i: (i, 0)),
            pl.BlockSpec((_BN, FP), lambda i: (i, 0)),
            pl.BlockSpec((_BN, F_IN), lambda i: (i, 0)),
            pl.BlockSpec((_BN, 1), lambda i: (i, 0)),
            full((F_IN, HD)), full((1, HD)), full((2 * HD, HD)), full((1, HD)),
            full((F_IN, HD)), full((1, HD)), full((2 * HD, HD)), full((1, HD)),
            full((2 * HD, HD)), full((1, HD)),
        ],
        out_specs=[
            pl.BlockSpec((_BN, HD), lambda i: (i, 0)),
            pl.BlockSpec((_BN, HD), lambda i: (i, 0)),
        ],
        out_shape=[
            jax.ShapeDtypeStruct((N, HD), f32),
            jax.ShapeDtypeStruct((N, HD), f32),
        ],
    )(s0, s1, x2, dinv, wz, bz, lzw, lzb, wh, bh, lhw, lhb, m1w, m1b)


_BE = 2000

def _tcc_body(gs, m2w, m2b, out):
    r = jnp.maximum(gs[...], 0.0)
    out[...] = jnp.dot(r, m2w[...],
                       precision=jax.lax.Precision.HIGHEST) + m2b[...]


def _tc_c(gs, m2w, m2b):
    grid = E // _BE
    return pl.pallas_call(
        _tcc_body,
        grid=(grid,),
        in_specs=[
            pl.BlockSpec((_BE, HD), lambda i: (i, 0)),
            pl.BlockSpec((HD, 2), lambda i: (0, 0)),
            pl.BlockSpec((1, 2), lambda i: (0, 0)),
        ],
        out_specs=pl.BlockSpec((_BE, 2), lambda i: (i, 0)),
        out_shape=jax.ShapeDtypeStruct((E, 2), f32),
    )(gs, m2w, m2b)


def kernel(x, edge_index, edge_weight, edge_src, edge_dst,
           W_z, b_z, Lz_W, Lz_b, W_r, b_r, Lr_W, Lr_b, W_h, b_h, Lh_W, Lh_b,
           att, M1_W, M1_b, M2_W, M2_b):
    x2 = x[0, :, :, 0]
    src = edge_index[0]
    dst = edge_index[1]

    pad = EPAD - E
    zpad_i = jnp.zeros((pad,), i32)
    src2d = jnp.concatenate([src, zpad_i]).reshape(CPAD, K)
    dst2d = jnp.concatenate([dst, jnp.full((pad,), N, i32)]).reshape(CPAD, K)
    ew2d = jnp.concatenate([edge_weight,
                            jnp.zeros((pad,), f32)]).reshape(CPAD, K)
    esrc2d = jnp.concatenate([edge_src, zpad_i]).reshape(CPAD, K)
    edst2d = jnp.concatenate([edge_dst, zpad_i]).reshape(CPAD, K)

    p0, p1 = _sc_deg(dst2d, ew2d)
    dinv, y0, y1 = _tc_a(p0.reshape(N, 1), p1.reshape(N, 1), x2)
    s0, s1 = _sc_rows(src2d, dst2d, ew2d, y0, y1)
    a_tab, b_tab = _tc_b(
        s0, s1, x2, dinv,
        W_z, b_z.reshape(1, HD), Lz_W, Lz_b.reshape(1, HD),
        W_h, b_h.reshape(1, HD), Lh_W, Lh_b.reshape(1, HD),
        M1_W, M1_b.reshape(1, HD))
    gs = _sc_edges(esrc2d, edst2d, a_tab, b_tab)
    return _tc_c(gs[:E], M2_W, M2_b.reshape(1, 2))


# speedup vs baseline: 1.2611x; 1.0730x over previous
import functools

import jax
import jax.numpy as jnp
from jax import lax
from jax.experimental import pallas as pl
from jax.experimental.pallas import tpu as pltpu
from jax.experimental.pallas import tpu_sc as plsc

N = 50000
E = 800000
F_IN = 80
FH = 40
FP = 64
HD = 64

NC = 2
NS = 16
NW = NC * NS
K = 128
CPAD = 6400
EPAD = CPAD * K
CH = CPAD // 2
EH = CH * K
IB2 = 4
IB3 = 8
IB3E = 4

ROWS_T = 3200
ROWS_LAST = N - 15 * ROWS_T
ZR = 400

f32 = jnp.float32
bf16 = jnp.bfloat16
i32 = jnp.int32

_MESH = plsc.VectorSubcoreMesh(core_axis_name="c", subcore_axis_name="s")
_SC_PARAMS = pltpu.CompilerParams(use_tc_tiling_on_sc=False,
                                  needs_layout_passes=False)


def _zero_vec_buf(ref, nelem):
    def body(g, carry):
        ref[pl.ds(g * 16, 16)] = jnp.zeros((16,), f32)
        return carry
    lax.fori_loop(0, nelem // 16, body, 0)


def _zero_row_buf(ref, nrows):
    def body(r, carry):
        z = jnp.zeros((32,), bf16)
        ref[r, pl.ds(0, 32)] = z
        ref[r, pl.ds(32, 32)] = z
        return carry
    lax.fori_loop(0, nrows, body, 0)


@functools.partial(
    pl.kernel,
    out_type=(jax.ShapeDtypeStruct((N,), f32),
              jax.ShapeDtypeStruct((N,), f32)),
    mesh=_MESH,
    compiler_params=_SC_PARAMS,
    scratch_types=[
        pltpu.VMEM((IB3, K), i32),
        pltpu.VMEM((IB3, K), f32),
        pltpu.VMEM((ZR,), f32),
        pltpu.VMEM_SHARED((N + 16,), f32),
        pltpu.SemaphoreType.DMA((IB3,)),
    ],
)
def _sc_deg(dst_hbm, ew_hbm, p0_hbm, p1_hbm, idx_b, ew_b, zbuf, deg_sh, sems):
    c = lax.axis_index("c")
    s = lax.axis_index("s")
    w = s * NC + c

    _zero_vec_buf(zbuf, ZR)

    @pl.when(s < 15)
    def _():
        for r in range(ROWS_T // ZR):
            pltpu.sync_copy(zbuf, deg_sh.at[pl.ds(s * ROWS_T + r * ZR, ZR)])

    @pl.when(s == 15)
    def _():
        for r in range(ROWS_LAST // ZR):
            pltpu.sync_copy(zbuf, deg_sh.at[pl.ds(15 * ROWS_T + r * ZR, ZR)])

    plsc.subcore_barrier()

    def body(b, carry):
        base = (w + NW * b) * IB3
        pltpu.sync_copy(dst_hbm.at[pl.ds(base, IB3)], idx_b)
        pltpu.sync_copy(ew_hbm.at[pl.ds(base, IB3)], ew_b)
        descs = [
            pltpu.async_copy(ew_b.at[t], deg_sh.at[idx_b.at[t]],
                             sems.at[t], add=True)
            for t in range(IB3)
        ]
        for d in descs:
            d.wait()
        return carry

    lax.fori_loop(0, CPAD // (NW * IB3), body, 0)
    plsc.subcore_barrier()

    def writeout(out_hbm):
        def wchunk(base):
            pltpu.sync_copy(deg_sh.at[pl.ds(base, ZR)], zbuf)
            pltpu.sync_copy(zbuf, out_hbm.at[pl.ds(base, ZR)])

        @pl.when(s < 15)
        def _():
            for r in range(ROWS_T // ZR):
                wchunk(s * ROWS_T + r * ZR)

        @pl.when(s == 15)
        def _():
            for r in range(ROWS_LAST // ZR):
                wchunk(15 * ROWS_T + r * ZR)

    @pl.when(c == 0)
    def _():
        writeout(p0_hbm)

    @pl.when(c == 1)
    def _():
        writeout(p1_hbm)


@functools.partial(
    pl.kernel,
    out_type=(jax.ShapeDtypeStruct((N, FP), bf16),
              jax.ShapeDtypeStruct((N, FP), bf16)),
    mesh=_MESH,
    compiler_params=_SC_PARAMS,
    scratch_types=[
        pltpu.VMEM((IB2, K), i32),
        pltpu.VMEM((IB2, K), i32),
        pltpu.VMEM((IB2, K), f32),
        pltpu.VMEM((IB2, K, FP), bf16),
        pltpu.VMEM((ZR, FP), bf16),
        pltpu.VMEM_SHARED((N + 16, FP), bf16),
        pltpu.SemaphoreType.DMA((IB2,)),
        pltpu.SemaphoreType.DMA((IB2,)),
    ],
)
def _sc_rows(src_hbm, dst_hbm, ew_hbm, y0_hbm, y1_hbm, s0_hbm, s1_hbm,
             isrc_b, idst_b, ew_b, rows, zrows, s_sh, semg, semsc):
    c = lax.axis_index("c")
    s = lax.axis_index("s")

    _zero_row_buf(zrows, ZR)

    @pl.when(s < 15)
    def _():
        for r in range(ROWS_T // ZR):
            pltpu.sync_copy(zrows, s_sh.at[pl.ds(s * ROWS_T + r * ZR, ZR)])

    @pl.when(s == 15)
    def _():
        for r in range(ROWS_LAST // ZR):
            pltpu.sync_copy(zrows, s_sh.at[pl.ds(15 * ROWS_T + r * ZR, ZR)])

    plsc.subcore_barrier()

    def pipeline(y_hbm):
        def body(b, carry):
            base = (s + NS * b) * IB2
            pltpu.sync_copy(src_hbm.at[pl.ds(base, IB2)], isrc_b)
            pltpu.sync_copy(dst_hbm.at[pl.ds(base, IB2)], idst_b)
            pltpu.sync_copy(ew_hbm.at[pl.ds(base, IB2)], ew_b)
            gd = [
                pltpu.async_copy(y_hbm.at[isrc_b.at[t]], rows.at[t],
                                 semg.at[t])
                for t in range(IB2)
            ]
            sd = []
            for t in range(IB2):
                gd[t].wait()

                def scale(k, carry2, t=t):
                    w16 = plsc.load_gather(
                        ew_b, [jnp.full((16,), t, i32),
                               jnp.full((16,), k, i32)])
                    wb = plsc.pack(w16, w16,
                                   format=plsc.PackFormat.INTERLEAVED)
                    rows[t, k, pl.ds(0, 32)] = rows[t, k, pl.ds(0, 32)] * wb
                    rows[t, k, pl.ds(32, 32)] = rows[t, k, pl.ds(32, 32)] * wb
                    return carry2

                lax.fori_loop(0, K, scale, 0)
                sd.append(pltpu.async_copy(
                    rows.at[t], s_sh.at[idst_b.at[t]], semsc.at[t], add=True))
            for d in sd:
                d.wait()
            return carry

        lax.fori_loop(0, CPAD // (NS * IB2), body, 0)

    @pl.when(c == 0)
    def _():
        pipeline(y0_hbm)

    @pl.when(c == 1)
    def _():
        pipeline(y1_hbm)

    plsc.subcore_barrier()

    def writeout(out_hbm):
        def wchunk(base):
            pltpu.sync_copy(s_sh.at[pl.ds(base, ZR)], zrows)
            pltpu.sync_copy(zrows, out_hbm.at[pl.ds(base, ZR)])

        @pl.when(s < 15)
        def _():
            for r in range(ROWS_T // ZR):
                wchunk(s * ROWS_T + r * ZR)

        @pl.when(s == 15)
        def _():
            for r in range(ROWS_LAST // ZR):
                wchunk(15 * ROWS_T + r * ZR)

    @pl.when(c == 0)
    def _():
        writeout(s0_hbm)

    @pl.when(c == 1)
    def _():
        writeout(s1_hbm)


@functools.partial(
    pl.kernel,
    out_type=jax.ShapeDtypeStruct((EH, HD), f32),
    mesh=_MESH,
    compiler_params=_SC_PARAMS,
    scratch_types=[
        pltpu.VMEM((IB3E, K), i32),
        pltpu.VMEM((IB3E, K), i32),
        pltpu.VMEM((K, HD), f32),
    ],
)
def _sc_edges(src_hbm, dst_hbm, a_hbm, b_hbm, gs_hbm, isrc_b, idst_b, rows):
    c = lax.axis_index("c")
    s = lax.axis_index("s")
    w = s * NC + c

    def body(b, carry):
        base = (w + NW * b) * IB3E
        pltpu.sync_copy(src_hbm.at[pl.ds(base, IB3E)], isrc_b)
        pltpu.sync_copy(dst_hbm.at[pl.ds(base, IB3E)], idst_b)
        for t in range(IB3E):
            pltpu.sync_copy(a_hbm.at[isrc_b.at[t]], rows)
            pltpu.sync_copy(b_hbm.at[idst_b.at[t]], rows, add=True)
            pltpu.sync_copy(rows, gs_hbm.at[pl.ds((base + t) * K, K)])
        return carry

    lax.fori_loop(0, CH // (NW * IB3E), body, 0)


_BN = 2000

def _tca_body(p0, p1, x2, dinv, y0, y1):
    deg = p0[...] + p1[...] + 1.0
    dv = lax.rsqrt(deg)
    dinv[...] = dv
    zpad = jnp.zeros((_BN, FP - FH), bf16)
    y0[...] = jnp.concatenate([(dv * x2[:, :FH]).astype(bf16), zpad], axis=1)
    y1[...] = jnp.concatenate([(dv * x2[:, FH:]).astype(bf16), zpad], axis=1)


def _tc_a(p0, p1, x2):
    grid = N // _BN
    return pl.pallas_call(
        _tca_body,
        grid=(grid,),
        in_specs=[
            pl.BlockSpec((_BN, 1), lambda i: (i, 0)),
            pl.BlockSpec((_BN, 1), lambda i: (i, 0)),
            pl.BlockSpec((_BN, F_IN), lambda i: (i, 0)),
        ],
        out_specs=[
            pl.BlockSpec((_BN, 1), lambda i: (i, 0)),
            pl.BlockSpec((_BN, FP), lambda i: (i, 0)),
            pl.BlockSpec((_BN, FP), lambda i: (i, 0)),
        ],
        out_shape=[
            jax.ShapeDtypeStruct((N, 1), f32),
            jax.ShapeDtypeStruct((N, FP), bf16),
            jax.ShapeDtypeStruct((N, FP), bf16),
        ],
    )(p0, p1, x2)


def _tcb_body(s0, s1, x2, dinv, wz, bz, lzw, lzb, wh, bh, lhw, lhb,
              m1w, m1b, a_out, b_out):
    dv = dinv[...]
    dv2 = dv * dv
    ax0 = dv * s0[:, :FH].astype(f32) + dv2 * x2[:, :FH]
    ax1 = dv * s1[:, :FH].astype(f32) + dv2 * x2[:, FH:]
    aggx = jnp.concatenate([ax0, ax1], axis=1)
    hp = jax.lax.Precision.HIGHEST
    wzp = jnp.dot(wz[...], lzw[:HD, :], precision=hp)
    bzp = jnp.dot(bz[...], lzw[:HD, :], precision=hp) + lzb[...]
    whp = jnp.dot(wh[...], lhw[:HD, :], precision=hp)
    bhp = jnp.dot(bh[...], lhw[:HD, :], precision=hp) + lhb[...]
    z = jax.nn.sigmoid(jnp.dot(aggx, wzp, precision=hp) + bzp)
    ht = jnp.tanh(jnp.dot(aggx, whp, precision=hp) + bhp)
    h = (1.0 - z) * ht
    a_out[...] = jnp.dot(h, m1w[:HD, :], precision=hp) + m1b[...]
    b_out[...] = jnp.dot(h, m1w[HD:, :], precision=hp)


def _tc_b(s0, s1, x2, dinv, wz, bz, lzw, lzb, wh, bh, lhw, lhb, m1w, m1b):
    grid = N // _BN
    full = lambda shape: pl.BlockSpec(shape, lambda i: tuple(0 for _ in shape))
    return pl.pallas_call(
        _tcb_body,
        grid=(grid,),
        in_specs=[
            pl.BlockSpec((_BN, FP), lambda i: (i, 0)),
            pl.BlockSpec((_BN, FP), lambda i: (i, 0)),
            pl.BlockSpec((_BN, F_IN), lambda i: (i, 0)),
            pl.BlockSpec((_BN, 1), lambda i: (i, 0)),
            full((F_IN, HD)), full((1, HD)), full((2 * HD, HD)), full((1, HD)),
            full((F_IN, HD)), full((1, HD)), full((2 * HD, HD)), full((1, HD)),
            full((2 * HD, HD)), full((1, HD)),
        ],
        out_specs=[
            pl.BlockSpec((_BN, HD), lambda i: (i, 0)),
            pl.BlockSpec((_BN, HD), lambda i: (i, 0)),
        ],
        out_shape=[
            jax.ShapeDtypeStruct((N, HD), f32),
            jax.ShapeDtypeStruct((N, HD), f32),
        ],
    )(s0, s1, x2, dinv, wz, bz, lzw, lzb, wh, bh, lhw, lhb, m1w, m1b)


_BE = 1600

def _tcc_body(gs, m2w, m2b, out):
    r = jnp.maximum(gs[...], 0.0)
    out[...] = jnp.dot(r, m2w[...],
                       precision=jax.lax.Precision.HIGHEST) + m2b[...]


def _tc_c(gs, m2w, m2b):
    ne = gs.shape[0]
    grid = ne // _BE
    return pl.pallas_call(
        _tcc_body,
        grid=(grid,),
        in_specs=[
            pl.BlockSpec((_BE, HD), lambda i: (i, 0)),
            pl.BlockSpec((HD, 2), lambda i: (0, 0)),
            pl.BlockSpec((1, 2), lambda i: (0, 0)),
        ],
        out_specs=pl.BlockSpec((_BE, 2), lambda i: (i, 0)),
        out_shape=jax.ShapeDtypeStruct((ne, 2), f32),
    )(gs, m2w, m2b)


def kernel(x, edge_index, edge_weight, edge_src, edge_dst,
           W_z, b_z, Lz_W, Lz_b, W_r, b_r, Lr_W, Lr_b, W_h, b_h, Lh_W, Lh_b,
           att, M1_W, M1_b, M2_W, M2_b):
    x2 = x[0, :, :, 0]
    src = edge_index[0]
    dst = edge_index[1]

    pad = EPAD - E
    zpad_i = jnp.zeros((pad,), i32)
    src2d = jnp.concatenate([src, zpad_i]).reshape(CPAD, K)
    dst2d = jnp.concatenate([dst, jnp.full((pad,), N, i32)]).reshape(CPAD, K)
    ew2d = jnp.concatenate([edge_weight,
                            jnp.zeros((pad,), f32)]).reshape(CPAD, K)
    esrc2d = jnp.concatenate([edge_src, zpad_i]).reshape(CPAD, K)
    edst2d = jnp.concatenate([edge_dst, zpad_i]).reshape(CPAD, K)

    p0, p1 = _sc_deg(dst2d, ew2d)
    dinv, y0, y1 = _tc_a(p0.reshape(N, 1), p1.reshape(N, 1), x2)
    s0, s1 = _sc_rows(src2d, dst2d, ew2d, y0, y1)
    a_tab, b_tab = _tc_b(
        s0, s1, x2, dinv,
        W_z, b_z.reshape(1, HD), Lz_W, Lz_b.reshape(1, HD),
        W_h, b_h.reshape(1, HD), Lh_W, Lh_b.reshape(1, HD),
        M1_W, M1_b.reshape(1, HD))
    m2b2 = M2_b.reshape(1, 2)
    gs_a = _sc_edges(esrc2d[:CH], edst2d[:CH], a_tab, b_tab)
    gs_b = _sc_edges(esrc2d[CH:], edst2d[CH:], a_tab, b_tab)
    out_a = _tc_c(gs_a, M2_W, m2b2)
    out_b = _tc_c(gs_b[:E - EH], M2_W, m2b2)
    return jnp.concatenate([out_a, out_b], axis=0)
